# baseline (speedup 1.0000x reference)
import functools
import math

import jax
import jax.numpy as jnp
import numpy as np
from jax import lax
from jax.experimental import pallas as pl
from jax.experimental.pallas import tpu as pltpu
from jax.experimental.pallas import tpu_sc as plsc

_SIZE = 100000
_PAD = 0
_SMOOTH = 0.1
_CONF = 1.0 - _SMOOTH
_EPS = float(np.float32(_SMOOTH / (_SIZE - 2)))
_C = _CONF * math.log(_CONF) + (_SIZE - 2) * _EPS * math.log(_EPS)

_NW = 32
_RPW = 512 // _NW


def _sc_gather_kernel(x_hbm, t_hbm, out_hbm, tgt_v, chunk_v, res_v, sem):
    c = lax.axis_index("c")
    s = lax.axis_index("s")
    wid = c * 16 + s
    base = wid * _RPW
    pltpu.sync_copy(t_hbm.at[pl.ds(base, _RPW)], tgt_v)
    tv = tgt_v[...]
    lane = lax.broadcasted_iota(jnp.int32, (16,), 0)
    handles = []
    for i in range(_RPW):
        ti = tv[i]
        ta = pl.multiple_of((ti >> 7) << 7, 128)
        ra = pl.multiple_of(((base + i) >> 3) << 3, 8)
        handles.append(
            pltpu.async_copy(x_hbm.at[pl.ds(ra, 8), pl.ds(ta, 128)],
                             chunk_v.at[pl.ds(i * 8, 8), :], sem))
    for h in handles:
        h.wait()
    vals = plsc.load_gather(chunk_v, [lane * 8 + ((base + lane) & 7), tv & 127])
    res_v[...] = jnp.where(tv != 0, vals, 0.0)
    pltpu.sync_copy(res_v, out_hbm.at[wid])


@functools.partial(
    pl.kernel,
    mesh=plsc.VectorSubcoreMesh(core_axis_name="c", subcore_axis_name="s"),
    out_type=jax.ShapeDtypeStruct((_NW, _RPW), jnp.float32),
    scratch_types=[
        pltpu.VMEM((_RPW,), jnp.int32),
        pltpu.VMEM((_RPW * 8, 128), jnp.float32),
        pltpu.VMEM((_RPW,), jnp.float32),
        pltpu.SemaphoreType.DMA,
    ],
    compiler_params=pltpu.CompilerParams(
        needs_layout_passes=False, skip_device_barrier=True),
)
def _sc_gather(x_hbm, t_hbm, out_hbm, tgt_v, chunk_v, res_v, sem):
    _sc_gather_kernel(x_hbm, t_hbm, out_hbm, tgt_v, chunk_v, res_v, sem)


def _ls_kernel(t_ref, x_ref, o_ref, acc_ref):
    b = pl.program_id(0)
    nb = pl.num_programs(0)
    R = x_ref.shape[1]

    @pl.when(b == 0)
    def _init():
        acc_ref[0] = 0.0
        acc_ref[1] = 0.0
        acc_ref[2] = 0.0

    x = x_ref[0]
    t = t_ref[0]
    m = (t != 0).astype(jnp.float32)

    S = jnp.sum(x, axis=1, keepdims=True)
    x0 = x[:, 0:1]
    acc_ref[0] += jnp.sum(m * (S - x0))

    rows = jax.lax.broadcasted_iota(jnp.int32, t.shape, 0) + b * R
    acc_ref[1] += jnp.sum(m)
    acc_ref[2] += jnp.sum(jnp.where((rows & 31) != 0, m, 0.0))

    @pl.when(b == nb - 1)
    def _fin():
        o_ref[0, 0] = acc_ref[0]
        o_ref[0, 1] = acc_ref[1]
        o_ref[0, 2] = acc_ref[2]


_ROWS_PER_BLK = 64


@jax.jit
def _label_smoothing_loss(x, target):
    B, S, V = x.shape
    R = _ROWS_PER_BLK
    nblk = (B * S) // R
    x2 = x.reshape(B * S, V)
    tflat = target.reshape(B * S)
    gpart = _sc_gather(x2, tflat)

    x3 = x.reshape(nblk, R, V)
    t3 = target.reshape(nblk, R, 1)
    out = pl.pallas_call(
        _ls_kernel,
        grid=(nblk,),
        in_specs=[
            pl.BlockSpec((1, R, 1), lambda b: (b, 0, 0)),
            pl.BlockSpec((1, R, V), lambda b: (b, 0, 0)),
        ],
        out_specs=pl.BlockSpec(memory_space=pltpu.SMEM),
        out_shape=jax.ShapeDtypeStruct((1, 3), jnp.float32),
        scratch_shapes=[pltpu.SMEM((3,), jnp.float32)],
        compiler_params=pltpu.CompilerParams(
            dimension_semantics=("arbitrary",),
        ),
    )(t3, x3)
    g = jnp.sum(gpart)
    kl = (out[0, 1] * np.float32(_C)
          - np.float32(_EPS) * out[0, 0]
          - np.float32(_CONF - _EPS) * g)
    return kl / out[0, 2]


def kernel(x, target):
    return _label_smoothing_loss(x, target)


# speedup vs baseline: 1.0074x; 1.0074x over previous
import functools
import math

import jax
import jax.numpy as jnp
import numpy as np
from jax import lax
from jax.experimental import pallas as pl
from jax.experimental.pallas import tpu as pltpu
from jax.experimental.pallas import tpu_sc as plsc

_SIZE = 100000
_PAD = 0
_SMOOTH = 0.1
_CONF = 1.0 - _SMOOTH
_EPS = float(np.float32(_SMOOTH / (_SIZE - 2)))
_C = _CONF * math.log(_CONF) + (_SIZE - 2) * _EPS * math.log(_EPS)

_NW = 32
_RPW = 512 // _NW


def _sc_gather_kernel(x_hbm, t_hbm, out_hbm, tgt_v, chunk_v, res_v, sem):
    c = lax.axis_index("c")
    s = lax.axis_index("s")
    wid = c * 16 + s
    base = wid * _RPW
    pltpu.sync_copy(t_hbm.at[pl.ds(base, _RPW)], tgt_v)
    tv = tgt_v[...]
    lane = lax.broadcasted_iota(jnp.int32, (16,), 0)
    handles = []
    for i in range(_RPW):
        ti = tv[i]
        ta = pl.multiple_of((ti >> 7) << 7, 128)
        ra = pl.multiple_of(((base + i) >> 3) << 3, 8)
        handles.append(
            pltpu.async_copy(x_hbm.at[pl.ds(ra, 8), pl.ds(ta, 128)],
                             chunk_v.at[pl.ds(i * 8, 8), :], sem))
    for h in handles:
        h.wait()
    vals = plsc.load_gather(chunk_v, [lane * 8 + ((base + lane) & 7), tv & 127])
    res_v[...] = jnp.where(tv != 0, vals, 0.0)
    pltpu.sync_copy(res_v, out_hbm.at[wid])


@functools.partial(
    pl.kernel,
    mesh=plsc.VectorSubcoreMesh(core_axis_name="c", subcore_axis_name="s"),
    out_type=jax.ShapeDtypeStruct((_NW, _RPW), jnp.float32),
    scratch_types=[
        pltpu.VMEM((_RPW,), jnp.int32),
        pltpu.VMEM((_RPW * 8, 128), jnp.float32),
        pltpu.VMEM((_RPW,), jnp.float32),
        pltpu.SemaphoreType.DMA,
    ],
    compiler_params=pltpu.CompilerParams(needs_layout_passes=False),
)
def _sc_gather(x_hbm, t_hbm, out_hbm, tgt_v, chunk_v, res_v, sem):
    _sc_gather_kernel(x_hbm, t_hbm, out_hbm, tgt_v, chunk_v, res_v, sem)


def _ls_kernel(t_ref, g_ref, x_ref, o_ref, acc_ref):
    b = pl.program_id(0)
    nb = pl.num_programs(0)
    R = x_ref.shape[1]

    @pl.when(b == 0)
    def _init():
        acc_ref[0] = 0.0
        acc_ref[1] = 0.0
        acc_ref[2] = 0.0

    x = x_ref[0]
    t = t_ref[0]
    m = (t != 0).astype(jnp.float32)

    S = jnp.sum(x, axis=1, keepdims=True)
    x0 = x[:, 0:1]
    acc_ref[0] += jnp.sum(m * (S - x0))

    rows = jax.lax.broadcasted_iota(jnp.int32, t.shape, 0) + b * R
    acc_ref[1] += jnp.sum(m)
    acc_ref[2] += jnp.sum(jnp.where((rows & 31) != 0, m, 0.0))

    @pl.when(b == nb - 1)
    def _fin():
        g = jnp.sum(g_ref[...])
        kl = (acc_ref[1] * np.float32(_C)
              - np.float32(_EPS) * acc_ref[0]
              - np.float32(_CONF - _EPS) * g)
        o_ref[0, 0] = kl / acc_ref[2]


_ROWS_PER_BLK = 64


@jax.jit
def _label_smoothing_loss(x, target):
    B, S, V = x.shape
    R = _ROWS_PER_BLK
    nblk = (B * S) // R
    x2 = x.reshape(B * S, V)
    tflat = target.reshape(B * S)
    gpart = _sc_gather(x2, tflat)

    x3 = x.reshape(nblk, R, V)
    t3 = target.reshape(nblk, R, 1)
    out = pl.pallas_call(
        _ls_kernel,
        grid=(nblk,),
        in_specs=[
            pl.BlockSpec((1, R, 1), lambda b: (b, 0, 0)),
            pl.BlockSpec((_NW, _RPW), lambda b: (0, 0)),
            pl.BlockSpec((1, R, V), lambda b: (b, 0, 0)),
        ],
        out_specs=pl.BlockSpec(memory_space=pltpu.SMEM),
        out_shape=jax.ShapeDtypeStruct((1, 1), jnp.float32),
        scratch_shapes=[pltpu.SMEM((3,), jnp.float32)],
        compiler_params=pltpu.CompilerParams(
            dimension_semantics=("arbitrary",),
        ),
    )(t3, gpart, x3)
    return out.reshape(())


def kernel(x, target):
    return _label_smoothing_loss(x, target)
